# Initial kernel scaffold; baseline (speedup 1.0000x reference)
import jax
import jax.numpy as jnp
from jax.experimental import pallas as pl


def kernel(x, edge_index, W1, b1, W2, b2, gumbel):
    raise NotImplementedError("write your pallas kernel here")


# speedup vs baseline: 24.4143x; 24.4143x over previous
import jax
import jax.numpy as jnp
from jax import lax
from jax.experimental import pallas as pl
from jax.experimental.pallas import tpu as pltpu
from jax.experimental.pallas import tpu_sc as plsc

N = 10000
E = 320000
F = 128
HID = 256
NC = 2
NS = 16
NW = NC * NS
EPW = E // NW
CHUNK = 125
NCHUNK = EPW // CHUNK
RPT = 1000
NDT = N // RPT

_MESH = plsc.VectorSubcoreMesh(core_axis_name="c", subcore_axis_name="s")


def _deg_body(dst_hbm, ones_hbm, zeros_hbm, out_hbm, dstv, onesv, stage, acc):
    c = lax.axis_index("c")
    s = lax.axis_index("s")
    wid = c * NS + s

    @pl.when(s < NDT)
    def _():
        pltpu.sync_copy(zeros_hbm, stage)
        pltpu.sync_copy(stage, acc.at[pl.ds(s * RPT, RPT)])

    pltpu.sync_copy(dst_hbm.at[wid], dstv)
    pltpu.sync_copy(ones_hbm, onesv)
    plsc.subcore_barrier()

    def body(j, carry):
        pltpu.sync_copy(onesv, acc.at[dstv.at[j]], add=True)
        return carry

    lax.fori_loop(0, NCHUNK, body, 0)
    plsc.subcore_barrier()

    @pl.when(s < NDT)
    def _():
        pltpu.sync_copy(acc.at[pl.ds(s * RPT, RPT)], stage)
        pltpu.sync_copy(stage, out_hbm.at[pl.ds(c * N + s * RPT, RPT)])


_deg_kernel = pl.kernel(
    _deg_body,
    out_type=jax.ShapeDtypeStruct((NC * N,), jnp.float32),
    mesh=_MESH,
    scratch_types=[
        pltpu.VMEM((NCHUNK, CHUNK), jnp.int32),
        pltpu.VMEM((CHUNK,), jnp.float32),
        pltpu.VMEM((RPT,), jnp.float32),
        pltpu.VMEM_SHARED((N,), jnp.float32),
    ],
)


SROWS = 40
NSTAGE = RPT // SROWS


def _edge_body(y_hbm, src_hbm, dst_hbm, zrows_hbm, out_hbm,
               srcv, dstv, buf, stage, acc, gsem):
    c = lax.axis_index("c")
    s = lax.axis_index("s")
    wid = c * NS + s

    @pl.when(s < NDT)
    def _():
        pltpu.sync_copy(zrows_hbm, stage)

        def zbody(k, carry):
            pltpu.sync_copy(stage, acc.at[pl.ds(s * RPT + k * SROWS, SROWS)])
            return carry

        lax.fori_loop(0, NSTAGE, zbody, 0)

    pltpu.sync_copy(src_hbm.at[wid], srcv)
    pltpu.sync_copy(dst_hbm.at[wid], dstv)
    plsc.subcore_barrier()

    def body(j, carry):
        pltpu.async_copy(y_hbm.at[srcv.at[j]], buf, gsem).wait()
        pltpu.sync_copy(buf, acc.at[dstv.at[j]], add=True)
        return carry

    lax.fori_loop(0, NCHUNK, body, 0)
    plsc.subcore_barrier()

    @pl.when(s < NDT)
    def _():
        def wbody(k, carry):
            pltpu.sync_copy(acc.at[pl.ds(s * RPT + k * SROWS, SROWS)], stage)
            pltpu.sync_copy(stage, out_hbm.at[c, pl.ds(s * RPT + k * SROWS, SROWS)])
            return carry

        lax.fori_loop(0, NSTAGE, wbody, 0)


_edge_kernel = pl.kernel(
    _edge_body,
    out_type=jax.ShapeDtypeStruct((NC, N, F), jnp.float32),
    mesh=_MESH,
    scratch_types=[
        pltpu.VMEM((NCHUNK, CHUNK), jnp.int32),
        pltpu.VMEM((NCHUNK, CHUNK), jnp.int32),
        pltpu.VMEM((CHUNK, F), jnp.float32),
        pltpu.VMEM((SROWS, F), jnp.float32),
        pltpu.VMEM_SHARED((N, F), jnp.float32),
        pltpu.SemaphoreType.DMA,
    ],
)


NB = 1000
_GRID = N // NB


def _prep_body(degp, x, y, dinv):
    d = degp[0] + degp[1] + 1.0
    di = lax.rsqrt(d)
    dinv[...] = di
    y[...] = x[...] * di


_prep = pl.pallas_call(
    _prep_body,
    grid=(_GRID,),
    in_specs=[
        pl.BlockSpec((NC, NB, 1), lambda i: (0, i, 0)),
        pl.BlockSpec((NB, F), lambda i: (i, 0)),
    ],
    out_specs=[
        pl.BlockSpec((NB, F), lambda i: (i, 0)),
        pl.BlockSpec((NB, 1), lambda i: (i, 0)),
    ],
    out_shape=[
        jax.ShapeDtypeStruct((N, F), jnp.float32),
        jax.ShapeDtypeStruct((N, 1), jnp.float32),
    ],
)


def _mid_body(part, y, dinv, W1, b1, W2, y2):
    p1 = dinv[...] * (part[0] + part[1] + y[...])
    h = jnp.dot(p1, W1[...], preferred_element_type=jnp.float32) + b1[...]
    z = jnp.dot(h, W2[...], preferred_element_type=jnp.float32)
    y2[...] = dinv[...] * z


_mid = pl.pallas_call(
    _mid_body,
    grid=(_GRID,),
    in_specs=[
        pl.BlockSpec((NC, NB, F), lambda i: (0, i, 0)),
        pl.BlockSpec((NB, F), lambda i: (i, 0)),
        pl.BlockSpec((NB, 1), lambda i: (i, 0)),
        pl.BlockSpec((F, HID), lambda i: (0, 0)),
        pl.BlockSpec((1, HID), lambda i: (0, 0)),
        pl.BlockSpec((HID, F), lambda i: (0, 0)),
    ],
    out_specs=pl.BlockSpec((NB, F), lambda i: (i, 0)),
    out_shape=jax.ShapeDtypeStruct((N, F), jnp.float32),
)


def _fin_body(part, y2, dinv, b2, g, out):
    emb = dinv[...] * (part[0] + part[1] + y2[...]) + b2[...]
    logit = emb + g[...]
    m = jnp.max(logit, axis=1, keepdims=True)
    col = lax.broadcasted_iota(jnp.int32, logit.shape, 1)
    first = jnp.min(jnp.where(logit >= m, col, F), axis=1, keepdims=True)
    out[...] = (col == first).astype(jnp.float32)


_fin = pl.pallas_call(
    _fin_body,
    grid=(_GRID,),
    in_specs=[
        pl.BlockSpec((NC, NB, F), lambda i: (0, i, 0)),
        pl.BlockSpec((NB, F), lambda i: (i, 0)),
        pl.BlockSpec((NB, 1), lambda i: (i, 0)),
        pl.BlockSpec((1, F), lambda i: (0, 0)),
        pl.BlockSpec((NB, F), lambda i: (i, 0)),
    ],
    out_specs=pl.BlockSpec((NB, F), lambda i: (i, 0)),
    out_shape=jax.ShapeDtypeStruct((N, F), jnp.float32),
)


def kernel(x, edge_index, W1, b1, W2, b2, gumbel):
    src = edge_index[0].reshape(NW, NCHUNK, CHUNK)
    dst = edge_index[1].reshape(NW, NCHUNK, CHUNK)
    ones = jnp.ones((CHUNK,), jnp.float32)
    zeros1 = jnp.zeros((RPT,), jnp.float32)
    zrows = jnp.zeros((SROWS, F), jnp.float32)

    degp = _deg_kernel(dst, ones, zeros1)
    y, dinv = _prep(degp.reshape(NC, N, 1), x)
    part1 = _edge_kernel(y, src, dst, zrows)
    y2 = _mid(part1, y, dinv, W1, b1.reshape(1, HID), W2)
    part2 = _edge_kernel(y2, src, dst, zrows)
    return _fin(part2, y2, dinv, b2.reshape(1, F), gumbel)
